# baseline (speedup 1.0000x reference)
import jax
import jax.numpy as jnp
from jax import lax
from jax.experimental import pallas as pl
from jax.experimental.pallas import tpu as pltpu
from jax.experimental.pallas import tpu_sc as plsc

N = 10000
D = 128
E = 320000
NC = 2
NS = 16
NW = NC * NS
CHUNK = 128
SUB = 4
NCHUNK = 80
EPAD = NW * NCHUNK * CHUNK
NPAD = 10240
ROWS_PER_TILE = NPAD // NS


def _sc_spmm(x, srcs, dsts, vals):
    mesh = plsc.VectorSubcoreMesh(core_axis_name="c", subcore_axis_name="s")

    def body(x_hbm, src_hbm, dst_hbm, val_hbm, out_hbm,
             acc_shared, src_all, dst_a, dst_b, val_a, val_b,
             rows_a, rows_b, sem_a, sem_b, sem_i):
        c = lax.axis_index("c")
        s = lax.axis_index("s")
        wid = c * NS + s

        pltpu.sync_copy(src_hbm.at[wid], src_all)

        zeros16 = jnp.zeros((16,), jnp.float32)

        def zero_row(e, carry):
            for j in range(D // 16):
                rows_a[e, pl.ds(j * 16, 16)] = zeros16
            return carry

        lax.fori_loop(0, CHUNK, zero_row, 0)
        for t in range(ROWS_PER_TILE // CHUNK):
            pltpu.sync_copy(rows_a,
                            acc_shared.at[pl.ds(s * ROWS_PER_TILE + t * CHUNK, CHUNK)])
        plsc.subcore_barrier()

        def issue(ci, rows, dstb, valb, sem):
            ci = jnp.minimum(ci, NCHUNK - 1)
            pltpu.async_copy(dst_hbm.at[wid, ci], dstb, sem)
            pltpu.async_copy(val_hbm.at[wid, ci], valb, sem)
            for k in range(SUB):
                g = CHUNK // SUB
                pltpu.async_copy(x_hbm.at[src_all.at[ci, pl.ds(k * g, g)]],
                                 rows.at[pl.ds(k * g, g)], sem)

        def wait(rows, dstb, valb, sem):
            pltpu.make_async_copy(dst_hbm.at[0, 0], dstb, sem).wait()
            pltpu.make_async_copy(val_hbm.at[0, 0], valb, sem).wait()
            pltpu.make_async_copy(x_hbm.at[src_all.at[0]], rows, sem).wait()

        def process(rows, dstb, valb):
            def scale(g, inner):
                base = g * 16
                vv = valb[pl.ds(base, 16)]
                for k in range(16):
                    vb = jnp.full((16,), vv[k], jnp.float32)
                    for j in range(D // 16):
                        sl = pl.ds(j * 16, 16)
                        rows[base + k, sl] = rows[base + k, sl] * vb
                return inner

            lax.fori_loop(0, CHUNK // 16, scale, 0)
            pltpu.sync_copy(rows, acc_shared.at[dstb], add=True)

        issue(0, rows_a, dst_a, val_a, sem_a)

        def pair(i, carry):
            ci = i * 2
            wait(rows_a, dst_a, val_a, sem_a)
            issue(ci + 1, rows_b, dst_b, val_b, sem_b)
            process(rows_a, dst_a, val_a)
            wait(rows_b, dst_b, val_b, sem_b)
            issue(ci + 2, rows_a, dst_a, val_a, sem_a)
            process(rows_b, dst_b, val_b)
            return carry

        lax.fori_loop(0, NCHUNK // 2, pair, 0)
        wait(rows_a, dst_a, val_a, sem_a)
        plsc.subcore_barrier()

        pltpu.sync_copy(acc_shared.at[pl.ds(s * ROWS_PER_TILE, ROWS_PER_TILE)],
                        out_hbm.at[c, pl.ds(s * ROWS_PER_TILE, ROWS_PER_TILE)])

    return pl.kernel(
        body,
        out_type=jax.ShapeDtypeStruct((NC, NPAD, D), jnp.float32),
        mesh=mesh,
        scratch_types=[
            pltpu.VMEM_SHARED((NPAD, D), jnp.float32),
            pltpu.VMEM((NCHUNK, CHUNK), jnp.int32),
            pltpu.VMEM((CHUNK,), jnp.int32),
            pltpu.VMEM((CHUNK,), jnp.int32),
            pltpu.VMEM((CHUNK,), jnp.float32),
            pltpu.VMEM((CHUNK,), jnp.float32),
            pltpu.VMEM((CHUNK, D), jnp.float32),
            pltpu.VMEM((CHUNK, D), jnp.float32),
            pltpu.SemaphoreType.DMA,
            pltpu.SemaphoreType.DMA,
            pltpu.SemaphoreType.DMA,
        ],
    )(x, srcs, dsts, vals)


def _tc_mix(p0, p1, i_1, w, beta_arr):
    BLK = 640

    def body(b_ref, p0_ref, p1_ref, i_ref, w_ref, o_ref):
        b = b_ref[0]
        m = (1.0 - b) * i_ref[...] + b * w_ref[...]
        o_ref[...] = jnp.dot(p0_ref[...] + p1_ref[...], m,
                             preferred_element_type=jnp.float32)

    return pl.pallas_call(
        body,
        grid=(NPAD // BLK,),
        in_specs=[
            pl.BlockSpec(memory_space=pltpu.SMEM),
            pl.BlockSpec((BLK, D), lambda i: (i, 0)),
            pl.BlockSpec((BLK, D), lambda i: (i, 0)),
            pl.BlockSpec((D, D), lambda i: (0, 0)),
            pl.BlockSpec((D, D), lambda i: (0, 0)),
        ],
        out_specs=pl.BlockSpec((BLK, D), lambda i: (i, 0)),
        out_shape=jax.ShapeDtypeStruct((NPAD, D), jnp.float32),
    )(beta_arr, p0, p1, i_1, w)


def kernel(X, edge_index, edge_vals, I_1, W, gamma, beta):
    src = edge_index[0].astype(jnp.int32)
    dst = edge_index[1].astype(jnp.int32)
    vals = edge_vals.astype(jnp.float32)
    pad = EPAD - E
    src = jnp.concatenate([src, jnp.zeros((pad,), jnp.int32)])
    dst = jnp.concatenate([dst, jnp.zeros((pad,), jnp.int32)])
    vals = jnp.concatenate([vals, jnp.zeros((pad,), jnp.float32)])
    srcs = src.reshape(NW, NCHUNK, CHUNK)
    dsts = dst.reshape(NW, NCHUNK, CHUNK)
    valsr = vals.reshape(NW, NCHUNK, CHUNK)

    partial = _sc_spmm(X, srcs, dsts, valsr)
    beta_arr = jnp.asarray(beta, jnp.float32).reshape(1)
    return _tc_mix(partial[0], partial[1], I_1, W, beta_arr)[:N]


# speedup vs baseline: 1.6969x; 1.6969x over previous
import jax
import jax.numpy as jnp
from jax import lax
from jax.experimental import pallas as pl
from jax.experimental.pallas import tpu as pltpu
from jax.experimental.pallas import tpu_sc as plsc

N = 10000
D = 128
E = 320000
NC = 2
NS = 16
NW = NC * NS
CHUNK = 128
NCHUNK = 80
EPAD = NW * NCHUNK * CHUNK
NPAD = 10240
ROWS_PER_TILE = NPAD // NS


def _sc_spmm(xbf, srcs, dsts, vals):
    mesh = plsc.VectorSubcoreMesh(core_axis_name="c", subcore_axis_name="s")

    def body(x_hbm, src_hbm, dst_hbm, val_hbm, out_hbm,
             acc_shared, src_a, src_b, dst_a, dst_b, val_a, val_b,
             rows_a, rows_b, rows_f,
             gsem_a, gsem_b, ssem_a, ssem_b, dsem_a, dsem_b):
        c = lax.axis_index("c")
        s = lax.axis_index("s")
        wid = c * NS + s

        zeros16 = jnp.zeros((16,), jnp.float32)

        def zero_row(e, carry):
            for j in range(D // 16):
                rows_f[e, pl.ds(j * 16, 16)] = zeros16
            return carry

        lax.fori_loop(0, CHUNK, zero_row, 0)
        for t in range(ROWS_PER_TILE // CHUNK):
            pltpu.sync_copy(rows_f,
                            acc_shared.at[pl.ds(s * ROWS_PER_TILE + t * CHUNK, CHUNK)])
        plsc.subcore_barrier()

        def fetch_src(ci, srcb, sem):
            ci = jnp.minimum(ci, NCHUNK - 1)
            pltpu.async_copy(src_hbm.at[wid, ci], srcb, sem)

        def wait_src(srcb, sem):
            pltpu.make_async_copy(src_hbm.at[0, 0], srcb, sem).wait()

        def fetch_dv(ci, dstb, valb, sem):
            ci = jnp.minimum(ci, NCHUNK - 1)
            pltpu.async_copy(dst_hbm.at[wid, ci], dstb, sem)
            pltpu.async_copy(val_hbm.at[wid, ci], valb, sem)

        def wait_dv(dstb, valb, sem):
            pltpu.make_async_copy(dst_hbm.at[0, 0], dstb, sem).wait()
            pltpu.make_async_copy(val_hbm.at[0, 0], valb, sem).wait()

        def issue_gather(srcb, rows, sem):
            pltpu.async_copy(x_hbm.at[srcb], rows, sem)

        def wait_gather(srcb, rows, sem):
            pltpu.make_async_copy(x_hbm.at[srcb], rows, sem).wait()

        def process(rows, dstb, valb):
            def scale(g, inner):
                base = g * 16
                vv = valb[pl.ds(base, 16)]
                for k in range(16):
                    e = base + k
                    vb = jnp.full((16,), vv[k], jnp.float32)
                    for j in range(D // 32):
                        w = rows[e, pl.ds(j * 16, 16)]
                        lo = lax.bitcast_convert_type(
                            lax.shift_left(w, 16), jnp.float32)
                        hi = lax.bitcast_convert_type(
                            lax.bitwise_and(w, jnp.int32(-65536)), jnp.float32)
                        rows_f[e, pl.ds(j * 32, 16)] = lo * vb
                        rows_f[e, pl.ds(j * 32 + 16, 16)] = hi * vb
                return inner

            lax.fori_loop(0, CHUNK // 16, scale, 0)
            pltpu.sync_copy(rows_f, acc_shared.at[dstb], add=True)

        sets = ((src_a, dst_a, val_a, rows_a, gsem_a, ssem_a, dsem_a),
                (src_b, dst_b, val_b, rows_b, gsem_b, ssem_b, dsem_b))

        fetch_src(0, src_a, ssem_a)
        fetch_dv(0, dst_a, val_a, dsem_a)
        wait_src(src_a, ssem_a)
        issue_gather(src_a, rows_a, gsem_a)
        fetch_src(1, src_b, ssem_b)
        fetch_dv(1, dst_b, val_b, dsem_b)

        def half(cj, S, T):
            (srcS, dstS, valS, rowsS, gsemS, ssemS, dsemS) = S
            (srcT, dstT, valT, rowsT, gsemT, ssemT, dsemT) = T
            wait_gather(srcS, rowsS, gsemS)
            wait_src(srcT, ssemT)
            issue_gather(srcT, rowsT, gsemT)
            fetch_src(cj + 2, srcS, ssemS)
            wait_dv(dstS, valS, dsemS)
            process(rowsS, dstS, valS)
            fetch_dv(cj + 2, dstS, valS, dsemS)

        def pair(i, carry):
            ci = i * 2
            half(ci, sets[0], sets[1])
            half(ci + 1, sets[1], sets[0])
            return carry

        lax.fori_loop(0, NCHUNK // 2, pair, 0)
        wait_gather(src_a, rows_a, gsem_a)
        wait_src(src_b, ssem_b)
        wait_dv(dst_a, val_a, dsem_a)
        wait_dv(dst_b, val_b, dsem_b)
        plsc.subcore_barrier()

        pltpu.sync_copy(acc_shared.at[pl.ds(s * ROWS_PER_TILE, ROWS_PER_TILE)],
                        out_hbm.at[c, pl.ds(s * ROWS_PER_TILE, ROWS_PER_TILE)])

    return pl.kernel(
        body,
        out_type=jax.ShapeDtypeStruct((NC, NPAD, D), jnp.float32),
        mesh=mesh,
        compiler_params=pltpu.CompilerParams(use_tc_tiling_on_sc=False),
        scratch_types=[
            pltpu.VMEM_SHARED((NPAD, D), jnp.float32),
            pltpu.VMEM((CHUNK,), jnp.int32),
            pltpu.VMEM((CHUNK,), jnp.int32),
            pltpu.VMEM((CHUNK,), jnp.int32),
            pltpu.VMEM((CHUNK,), jnp.int32),
            pltpu.VMEM((CHUNK,), jnp.float32),
            pltpu.VMEM((CHUNK,), jnp.float32),
            pltpu.VMEM((CHUNK, D // 2), jnp.int32),
            pltpu.VMEM((CHUNK, D // 2), jnp.int32),
            pltpu.VMEM((CHUNK, D), jnp.float32),
            pltpu.SemaphoreType.DMA,
            pltpu.SemaphoreType.DMA,
            pltpu.SemaphoreType.DMA,
            pltpu.SemaphoreType.DMA,
            pltpu.SemaphoreType.DMA,
            pltpu.SemaphoreType.DMA,
        ],
    )(xbf, srcs, dsts, vals)


def _tc_mix(p0, p1, i_1, w, beta_arr):
    BLK = 640

    def body(b_ref, p0_ref, p1_ref, i_ref, w_ref, o_ref):
        b = b_ref[0]
        m = (1.0 - b) * i_ref[...] + b * w_ref[...]
        o_ref[...] = jnp.dot(p0_ref[...] + p1_ref[...], m,
                             preferred_element_type=jnp.float32)

    return pl.pallas_call(
        body,
        grid=(NPAD // BLK,),
        in_specs=[
            pl.BlockSpec(memory_space=pltpu.SMEM),
            pl.BlockSpec((BLK, D), lambda i: (i, 0)),
            pl.BlockSpec((BLK, D), lambda i: (i, 0)),
            pl.BlockSpec((D, D), lambda i: (0, 0)),
            pl.BlockSpec((D, D), lambda i: (0, 0)),
        ],
        out_specs=pl.BlockSpec((BLK, D), lambda i: (i, 0)),
        out_shape=jax.ShapeDtypeStruct((NPAD, D), jnp.float32),
    )(beta_arr, p0, p1, i_1, w)


def kernel(X, edge_index, edge_vals, I_1, W, gamma, beta):
    src = edge_index[0].astype(jnp.int32)
    dst = edge_index[1].astype(jnp.int32)
    vals = edge_vals.astype(jnp.float32)
    pad = EPAD - E
    src = jnp.concatenate([src, jnp.zeros((pad,), jnp.int32)])
    dst = jnp.concatenate([dst, jnp.zeros((pad,), jnp.int32)])
    vals = jnp.concatenate([vals, jnp.zeros((pad,), jnp.float32)])
    srcs = src.reshape(NW, NCHUNK, CHUNK)
    dsts = dst.reshape(NW, NCHUNK, CHUNK)
    valsr = vals.reshape(NW, NCHUNK, CHUNK)

    xr = X.reshape(N, D // 32, 2, 16)
    xi = jnp.stack([xr[:, :, 0, :], xr[:, :, 1, :]], axis=-1)
    xbf = xi.reshape(N, D // 2, 2).astype(jnp.bfloat16)
    xbf = jax.lax.bitcast_convert_type(xbf, jnp.int32)

    partial = _sc_spmm(xbf, srcs, dsts, valsr)
    beta_arr = jnp.asarray(beta, jnp.float32).reshape(1)
    return _tc_mix(partial[0], partial[1], I_1, W, beta_arr)[:N]
